# baseline (speedup 1.0000x reference)
import functools

import jax
import jax.numpy as jnp
from jax import lax
from jax.experimental import pallas as pl
from jax.experimental.pallas import tpu as pltpu
from jax.experimental.pallas import tpu_sc as plsc

BATCH = 16384
FIELDS = 26
DIM = 64
PAD_DIM = 128
VOCAB = 1000001
VOCAB_PAD = 1000008

NC = 2
NS = 16
NW = NC * NS
L = 16
B = BATCH * FIELDS
B_PER_W = B // NW
ROWS_PER_W = BATCH // NW
CHUNK_ROWS = 16
CHUNK = CHUNK_ROWS * FIELDS
N_CHUNKS = ROWS_PER_W // CHUNK_ROWS

WB_ROWS = 256
WB_PER_W = 122
WT_ROWS = 24
WT_WORKERS = 24
WT_BASE = NW * WB_PER_W * WB_ROWS

_mesh = plsc.VectorSubcoreMesh(core_axis_name="c", subcore_axis_name="s")
_params = pltpu.CompilerParams(
    use_tc_tiling_on_sc=True, needs_layout_passes=False
)


@functools.partial(
    pl.kernel,
    mesh=_mesh,
    out_type=jax.ShapeDtypeStruct((VOCAB_PAD, PAD_DIM), jnp.float32),
    scratch_types=[
        pltpu.VMEM((WB_ROWS, DIM), jnp.float32),
        pltpu.VMEM((WB_ROWS, PAD_DIM), jnp.float32),
    ],
    compiler_params=_params,
)
def _widen_table_kernel(emb_hbm, tbl_hbm, narrow_v, wide_v):
    wid = lax.axis_index("s") * NC + lax.axis_index("c")

    def widen_block(r0, n_rows):
        pltpu.sync_copy(emb_hbm.at[pl.ds(r0, n_rows)], narrow_v.at[pl.ds(0, n_rows)])

        def row_body(r, carry):
            for c in range(0, DIM, L):
                wide_v[r, pl.ds(c, L)] = narrow_v[r, pl.ds(c, L)]
            return carry

        lax.fori_loop(0, n_rows, row_body, 0)
        pltpu.sync_copy(
            wide_v.at[pl.ds(0, n_rows)], tbl_hbm.at[pl.ds(r0, n_rows)]
        )

    base = wid * WB_PER_W * WB_ROWS

    def block_body(i, carry):
        widen_block(base + i * WB_ROWS, WB_ROWS)
        return carry

    lax.fori_loop(0, WB_PER_W, block_body, 0)

    @pl.when(wid < WT_WORKERS)
    def _tail():
        widen_block(WT_BASE + wid * WT_ROWS, WT_ROWS)


@functools.partial(
    pl.kernel,
    mesh=_mesh,
    out_type=jax.ShapeDtypeStruct((B, DIM), jnp.float32),
    scratch_types=[
        pltpu.VMEM((B_PER_W,), jnp.int32),
        pltpu.VMEM((CHUNK, PAD_DIM), jnp.float32),
        pltpu.SemaphoreType.DMA,
        pltpu.SemaphoreType.DMA,
    ],
    compiler_params=pltpu.CompilerParams(
        use_tc_tiling_on_sc=False, needs_layout_passes=False
    ),
)
def _gather_kernel(tbl_hbm, idx_hbm, out_hbm, idx_v, rows_v, gsem, wsem):
    wid = lax.axis_index("s") * NC + lax.axis_index("c")
    base = wid * B_PER_W
    pltpu.sync_copy(idx_hbm.at[pl.ds(base, B_PER_W)], idx_v)

    def chunk_body(i, carry):
        off = i * CHUNK
        pltpu.async_copy(
            tbl_hbm.at[idx_v.at[pl.ds(off, CHUNK)]], rows_v, gsem
        ).wait()
        pltpu.async_copy(
            rows_v.at[:, pl.ds(0, DIM)],
            out_hbm.at[pl.ds(base + off, CHUNK)],
            wsem,
        ).wait()
        return carry

    lax.fori_loop(0, N_CHUNKS, chunk_body, 0)


def kernel(x, emb):
    tbl = _widen_table_kernel(emb)
    out = _gather_kernel(tbl, x.reshape(-1).astype(jnp.int32))
    return out.reshape(BATCH, FIELDS, DIM)


# speedup vs baseline: 1.1596x; 1.1596x over previous
import functools

import jax
import jax.numpy as jnp
from jax import lax
from jax.experimental import pallas as pl
from jax.experimental.pallas import tpu as pltpu
from jax.experimental.pallas import tpu_sc as plsc

BATCH = 16384
FIELDS = 26
DIM = 64
PAD_DIM = 128
VOCAB = 1000001
VOCAB_PAD = 1000008

NC = 2
NS = 16
NW = NC * NS
L = 16
B = BATCH * FIELDS
B_PER_W = B // NW
ROWS_PER_W = BATCH // NW
CHUNK = 208
N_CHUNKS = B_PER_W // CHUNK

WB_ROWS = 256
WB_PAIRS = 61
WT_ROWS = 24
WT_WORKERS = 24
WT_BASE = NW * WB_PAIRS * 2 * WB_ROWS
ROW_UNROLL = 8

_mesh = plsc.VectorSubcoreMesh(core_axis_name="c", subcore_axis_name="s")


@functools.partial(
    pl.kernel,
    mesh=_mesh,
    out_type=jax.ShapeDtypeStruct((VOCAB_PAD, PAD_DIM), jnp.float32),
    scratch_types=[
        pltpu.VMEM((WB_ROWS, DIM), jnp.float32),
        pltpu.VMEM((WB_ROWS, PAD_DIM), jnp.float32),
        pltpu.VMEM((WB_ROWS, PAD_DIM), jnp.float32),
        pltpu.SemaphoreType.DMA,
        pltpu.SemaphoreType.DMA,
    ],
    compiler_params=pltpu.CompilerParams(
        use_tc_tiling_on_sc=True, needs_layout_passes=False
    ),
)
def _widen_table_kernel(emb_hbm, tbl_hbm, narrow_v, wide0_v, wide1_v, s0, s1):
    wid = lax.axis_index("s") * NC + lax.axis_index("c")
    base = wid * WB_PAIRS * 2 * WB_ROWS

    def widen_rows(wide_v, n_rows):
        def row_body(r8, carry):
            for dr in range(ROW_UNROLL):
                for c in range(0, DIM, L):
                    wide_v[r8 * ROW_UNROLL + dr, pl.ds(c, L)] = narrow_v[
                        r8 * ROW_UNROLL + dr, pl.ds(c, L)
                    ]
            return carry

        lax.fori_loop(0, n_rows // ROW_UNROLL, row_body, 0)

    def drain(wide_v, sem):
        pltpu.make_async_copy(
            wide_v, tbl_hbm.at[pl.ds(0, WB_ROWS)], sem
        ).wait()

    def pair_body(j, carry):
        r0 = base + j * 2 * WB_ROWS
        pltpu.sync_copy(emb_hbm.at[pl.ds(r0, WB_ROWS)], narrow_v)

        @pl.when(j > 0)
        def _():
            drain(wide0_v, s0)

        widen_rows(wide0_v, WB_ROWS)
        pltpu.async_copy(wide0_v, tbl_hbm.at[pl.ds(r0, WB_ROWS)], s0)
        r1 = r0 + WB_ROWS
        pltpu.sync_copy(emb_hbm.at[pl.ds(r1, WB_ROWS)], narrow_v)

        @pl.when(j > 0)
        def _():
            drain(wide1_v, s1)

        widen_rows(wide1_v, WB_ROWS)
        pltpu.async_copy(wide1_v, tbl_hbm.at[pl.ds(r1, WB_ROWS)], s1)
        return carry

    lax.fori_loop(0, WB_PAIRS, pair_body, 0)
    drain(wide0_v, s0)
    drain(wide1_v, s1)

    @pl.when(wid < WT_WORKERS)
    def _tail():
        t0 = WT_BASE + wid * WT_ROWS
        pltpu.sync_copy(
            emb_hbm.at[pl.ds(t0, WT_ROWS)], narrow_v.at[pl.ds(0, WT_ROWS)]
        )
        widen_rows(wide0_v, WT_ROWS)
        pltpu.sync_copy(
            wide0_v.at[pl.ds(0, WT_ROWS)], tbl_hbm.at[pl.ds(t0, WT_ROWS)]
        )


@functools.partial(
    pl.kernel,
    mesh=_mesh,
    out_type=jax.ShapeDtypeStruct((B, DIM), jnp.float32),
    scratch_types=[
        pltpu.VMEM((ROWS_PER_W, FIELDS), jnp.int32),
        pltpu.VMEM((B_PER_W,), jnp.int32),
        pltpu.VMEM((CHUNK, PAD_DIM), jnp.float32),
        pltpu.VMEM((CHUNK, PAD_DIM), jnp.float32),
        pltpu.SemaphoreType.DMA,
        pltpu.SemaphoreType.DMA,
        pltpu.SemaphoreType.DMA,
        pltpu.SemaphoreType.DMA,
    ],
    compiler_params=pltpu.CompilerParams(
        use_tc_tiling_on_sc=False, needs_layout_passes=False
    ),
)
def _gather_kernel(
    tbl_hbm, x_hbm, out_hbm, idx2d_v, idx_v, rows0_v, rows1_v, g0, g1, w0, w1
):
    wid = lax.axis_index("s") * NC + lax.axis_index("c")
    base = wid * B_PER_W
    row0 = wid * ROWS_PER_W
    pltpu.sync_copy(x_hbm.at[pl.ds(row0, ROWS_PER_W)], idx2d_v)

    lanes = lax.iota(jnp.int32, L)

    def flatten_body(k, carry):
        r, c = carry
        v = plsc.load_gather(idx2d_v, [r, c])
        idx_v[pl.ds(k * L, L)] = v
        c = c + L
        wrap = c >= FIELDS
        c = jnp.where(wrap, c - FIELDS, c)
        r = jnp.where(wrap, r + 1, r)
        return r, c

    lax.fori_loop(
        0, B_PER_W // L, flatten_body, (jnp.zeros(L, jnp.int32), lanes)
    )

    def gather(c, rows_v, gsem):
        pltpu.async_copy(
            tbl_hbm.at[idx_v.at[pl.ds(c * CHUNK, CHUNK)]], rows_v, gsem
        )

    def wait_gather(rows_v, gsem):
        pltpu.make_async_copy(
            tbl_hbm.at[pl.ds(0, CHUNK)], rows_v, gsem
        ).wait()

    def writeback(c, rows_v, wsem):
        pltpu.async_copy(
            rows_v.at[:, pl.ds(0, DIM)],
            out_hbm.at[pl.ds(base + c * CHUNK, CHUNK)],
            wsem,
        )

    def wait_writeback(rows_v, wsem):
        pltpu.make_async_copy(
            rows_v.at[:, pl.ds(0, DIM)],
            out_hbm.at[pl.ds(0, CHUNK)],
            wsem,
        ).wait()

    slots = ((rows0_v, g0, w0), (rows1_v, g1, w1))
    gather(0, rows0_v, g0)

    def chunk_pair(j, carry):
        for s in range(2):
            c = j * 2 + s
            rows_v, gsem, wsem = slots[s]
            o_rows, o_g, o_w = slots[1 - s]

            @pl.when(c > 0)
            def _():
                wait_writeback(o_rows, o_w)

            @pl.when(c + 1 < N_CHUNKS)
            def _():
                gather(c + 1, o_rows, o_g)

            wait_gather(rows_v, gsem)
            writeback(c, rows_v, wsem)
        return carry

    lax.fori_loop(0, N_CHUNKS // 2, chunk_pair, 0)
    wait_writeback(rows1_v, w1)


def kernel(x, emb):
    tbl = _widen_table_kernel(emb)
    out = _gather_kernel(tbl, x.astype(jnp.int32))
    return out.reshape(BATCH, FIELDS, DIM)


# speedup vs baseline: 1.1706x; 1.0095x over previous
import functools

import jax
import jax.numpy as jnp
from jax import lax
from jax.experimental import pallas as pl
from jax.experimental.pallas import tpu as pltpu
from jax.experimental.pallas import tpu_sc as plsc

BATCH = 16384
FIELDS = 26
DIM = 64
PAD_DIM = 128
VOCAB = 1000001
VOCAB_PAD = 1000008

NC = 2
NS = 16
NW = NC * NS
L = 16
B = BATCH * FIELDS
B_PER_W = B // NW
ROWS_PER_W = BATCH // NW
CHUNK_ROWS = 8
CHUNK = CHUNK_ROWS * FIELDS
N_CHUNKS = B_PER_W // CHUNK

XSB_ROWS = 64
XSB = ROWS_PER_W // XSB_ROWS
XSB_IDX = XSB_ROWS * FIELDS

WB_ROWS = 192
WB_PER_W = 162
WT_ROWS = 144
WT_BASE = NW * WB_PER_W * WB_ROWS
WT2_OFF = WT_BASE + NW * WT_ROWS
WT2_ROWS = 64
ROW_UNROLL = 8

_mesh = plsc.VectorSubcoreMesh(core_axis_name="c", subcore_axis_name="s")


@functools.partial(
    pl.kernel,
    mesh=_mesh,
    out_type=(
        jax.ShapeDtypeStruct((VOCAB_PAD, PAD_DIM), jnp.float32),
        jax.ShapeDtypeStruct((B,), jnp.int32),
    ),
    scratch_types=[
        pltpu.VMEM((XSB_ROWS, FIELDS), jnp.int32),
        pltpu.VMEM((XSB_IDX,), jnp.int32),
        pltpu.VMEM((WB_ROWS, DIM), jnp.float32),
        pltpu.VMEM((WB_ROWS, DIM), jnp.float32),
        pltpu.VMEM((WB_ROWS, PAD_DIM), jnp.float32),
        pltpu.VMEM((WB_ROWS, PAD_DIM), jnp.float32),
        pltpu.SemaphoreType.DMA,
        pltpu.SemaphoreType.DMA,
        pltpu.SemaphoreType.DMA,
        pltpu.SemaphoreType.DMA,
    ],
    compiler_params=pltpu.CompilerParams(
        use_tc_tiling_on_sc=True, needs_layout_passes=False
    ),
)
def _prep_kernel(
    emb_hbm, x_hbm, tbl_hbm, idx_hbm,
    xv, idxbuf, n0, n1, w0v, w1v, i0, i1, o0, o1,
):
    wid = lax.axis_index("s") * NC + lax.axis_index("c")

    xrow0 = wid * ROWS_PER_W
    xbase = wid * B_PER_W
    lanes = lax.iota(jnp.int32, L)

    for sb in range(XSB):
        pltpu.sync_copy(x_hbm.at[pl.ds(xrow0 + sb * XSB_ROWS, XSB_ROWS)], xv)

        def flatten_body(k, carry):
            r, c = carry
            v = plsc.load_gather(xv, [r, c])
            idxbuf[pl.ds(k * L, L)] = v
            c = c + L
            wrap = c >= FIELDS
            c = jnp.where(wrap, c - FIELDS, c)
            r = jnp.where(wrap, r + 1, r)
            return r, c

        lax.fori_loop(
            0, XSB_IDX // L, flatten_body, (jnp.zeros(L, jnp.int32), lanes)
        )
        pltpu.sync_copy(
            idxbuf, idx_hbm.at[pl.ds(xbase + sb * XSB_IDX, XSB_IDX)]
        )

    base = wid * WB_PER_W * WB_ROWS

    def widen_rows(narrow_v, wide_v, n_rows):
        def row_body(r8, carry):
            for dr in range(ROW_UNROLL):
                for c in range(0, DIM, L):
                    wide_v[r8 * ROW_UNROLL + dr, pl.ds(c, L)] = narrow_v[
                        r8 * ROW_UNROLL + dr, pl.ds(c, L)
                    ]
            return carry

        lax.fori_loop(0, n_rows // ROW_UNROLL, row_body, 0)

    def start_in(narrow_v, sem, blk):
        pltpu.async_copy(
            emb_hbm.at[pl.ds(base + blk * WB_ROWS, WB_ROWS)], narrow_v, sem
        )

    def wait_in(narrow_v, sem):
        pltpu.make_async_copy(
            emb_hbm.at[pl.ds(0, WB_ROWS)], narrow_v, sem
        ).wait()

    def start_out(wide_v, sem, blk):
        pltpu.async_copy(
            wide_v, tbl_hbm.at[pl.ds(base + blk * WB_ROWS, WB_ROWS)], sem
        )

    def wait_out(wide_v, sem):
        pltpu.make_async_copy(
            wide_v, tbl_hbm.at[pl.ds(0, WB_ROWS)], sem
        ).wait()

    start_in(n0, i0, 0)

    def pair_body(j, carry):
        wait_in(n0, i0)
        start_in(n1, i1, 2 * j + 1)

        @pl.when(j > 0)
        def _():
            wait_out(w0v, o0)

        widen_rows(n0, w0v, WB_ROWS)
        start_out(w0v, o0, 2 * j)
        wait_in(n1, i1)

        @pl.when(2 * j + 2 < WB_PER_W)
        def _():
            start_in(n0, i0, 2 * j + 2)

        @pl.when(j > 0)
        def _():
            wait_out(w1v, o1)

        widen_rows(n1, w1v, WB_ROWS)
        start_out(w1v, o1, 2 * j + 1)
        return carry

    lax.fori_loop(0, WB_PER_W // 2, pair_body, 0)
    wait_out(w0v, o0)
    wait_out(w1v, o1)

    def widen_tail(r0, n_rows):
        pltpu.sync_copy(
            emb_hbm.at[pl.ds(r0, n_rows)], n0.at[pl.ds(0, n_rows)]
        )
        widen_rows(n0, w0v, n_rows)
        pltpu.sync_copy(
            w0v.at[pl.ds(0, n_rows)], tbl_hbm.at[pl.ds(r0, n_rows)]
        )

    widen_tail(WT_BASE + wid * WT_ROWS, WT_ROWS)

    @pl.when(wid == 0)
    def _():
        widen_tail(WT2_OFF, WT2_ROWS)


@functools.partial(
    pl.kernel,
    mesh=_mesh,
    out_type=jax.ShapeDtypeStruct((BATCH, FIELDS, DIM), jnp.float32),
    scratch_types=[
        pltpu.VMEM((B_PER_W,), jnp.int32),
        pltpu.VMEM((CHUNK, PAD_DIM), jnp.float32),
        pltpu.VMEM((CHUNK, PAD_DIM), jnp.float32),
        pltpu.SemaphoreType.DMA,
        pltpu.SemaphoreType.DMA,
        pltpu.SemaphoreType.DMA,
        pltpu.SemaphoreType.DMA,
    ],
    compiler_params=pltpu.CompilerParams(
        use_tc_tiling_on_sc=False, needs_layout_passes=False
    ),
)
def _gather_kernel(
    tbl_hbm, idx_hbm, out_hbm, idx_v, rows0_v, rows1_v, g0, g1, w0, w1
):
    wid = lax.axis_index("s") * NC + lax.axis_index("c")
    base = wid * B_PER_W
    row0 = wid * ROWS_PER_W
    pltpu.sync_copy(idx_hbm.at[pl.ds(base, B_PER_W)], idx_v)

    def gather(c, rows_v, gsem):
        pltpu.async_copy(
            tbl_hbm.at[idx_v.at[pl.ds(c * CHUNK, CHUNK)]], rows_v, gsem
        )

    def wait_gather(rows_v, gsem):
        pltpu.make_async_copy(
            tbl_hbm.at[pl.ds(0, CHUNK)], rows_v, gsem
        ).wait()

    def writeback(c, rows_v, wsem):
        for k in range(CHUNK_ROWS):
            pltpu.async_copy(
                rows_v.at[pl.ds(k * FIELDS, FIELDS), pl.ds(0, DIM)],
                out_hbm.at[row0 + c * CHUNK_ROWS + k],
                wsem,
            )

    def wait_writeback(rows_v, wsem):
        for k in range(CHUNK_ROWS):
            pltpu.make_async_copy(
                rows_v.at[pl.ds(0, FIELDS), pl.ds(0, DIM)],
                out_hbm.at[0],
                wsem,
            ).wait()

    slots = ((rows0_v, g0, w0), (rows1_v, g1, w1))
    gather(0, rows0_v, g0)

    def chunk_pair(j, carry):
        for s in range(2):
            c = j * 2 + s
            rows_v, gsem, wsem = slots[s]
            o_rows, o_g, o_w = slots[1 - s]

            @pl.when(c > 0)
            def _():
                wait_writeback(o_rows, o_w)

            @pl.when(c + 1 < N_CHUNKS)
            def _():
                gather(c + 1, o_rows, o_g)

            wait_gather(rows_v, gsem)
            writeback(c, rows_v, wsem)
        return carry

    lax.fori_loop(0, N_CHUNKS // 2, chunk_pair, 0)
    wait_writeback(rows1_v, w1)


def kernel(x, emb):
    tbl, idx = _prep_kernel(emb, x.astype(jnp.int32))
    return _gather_kernel(tbl, idx)
